# Initial kernel scaffold; baseline (speedup 1.0000x reference)
import functools

import jax
import jax.numpy as jnp
from jax import lax
from jax.experimental import pallas as pl
from jax.experimental.pallas import tpu as pltpu
from jax.experimental.pallas import tpu_sc as plsc

N = 10000
E = 320000
D = 128

NC = 2
NS = 16
K = 80
EDGES_PER_TILE = E // (NC * NS)
CHUNKS_PER_TILE = EDGES_PER_TILE // K
ROWS_PER_TILE = N // NS
ZROWS = 125


def _sc_scatter_kernel(x_hbm, src_hbm, dst_hbm, out_hbm,
                       src_v, dst_v, rows_v, zero_v, acc_shared, gsem):
    c = lax.axis_index("c")
    s = lax.axis_index("s")
    chunk0 = (c * NS + s) * CHUNKS_PER_TILE

    pltpu.sync_copy(src_hbm.at[pl.ds(chunk0, CHUNKS_PER_TILE)], src_v)
    pltpu.sync_copy(dst_hbm.at[pl.ds(chunk0, CHUNKS_PER_TILE)], dst_v)

    def _zrow(i, carry):
        for j in range(D // 16):
            zero_v[i, pl.ds(j * 16, 16)] = jnp.zeros((16,), jnp.float32)
        return carry
    lax.fori_loop(0, ZROWS, _zrow, 0)
    for r in range(ROWS_PER_TILE // ZROWS):
        pltpu.sync_copy(
            zero_v, acc_shared.at[pl.ds(s * ROWS_PER_TILE + r * ZROWS, ZROWS)])
    plsc.subcore_barrier()

    def _chunk(j, carry):
        pltpu.async_copy(x_hbm.at[src_v.at[j]], rows_v, gsem).wait()
        pltpu.sync_copy(rows_v, acc_shared.at[dst_v.at[j]], add=True)
        return carry
    lax.fori_loop(0, CHUNKS_PER_TILE, _chunk, 0)
    plsc.subcore_barrier()

    pltpu.sync_copy(acc_shared.at[pl.ds(s * ROWS_PER_TILE, ROWS_PER_TILE)],
                    out_hbm.at[c, pl.ds(s * ROWS_PER_TILE, ROWS_PER_TILE)])


@jax.jit
def _sc_scatter(x, src2d, dst2d):
    return pl.kernel(
        _sc_scatter_kernel,
        out_type=jax.ShapeDtypeStruct((NC, N, D), jnp.float32),
        mesh=plsc.VectorSubcoreMesh(core_axis_name="c", subcore_axis_name="s"),
        scratch_types=[
            pltpu.VMEM((CHUNKS_PER_TILE, K), jnp.int32),
            pltpu.VMEM((CHUNKS_PER_TILE, K), jnp.int32),
            pltpu.VMEM((K, D), jnp.float32),
            pltpu.VMEM((ZROWS, D), jnp.float32),
            pltpu.VMEM_SHARED((N, D), jnp.float32),
            pltpu.SemaphoreType.DMA,
        ],
    )(x, src2d, dst2d)


def _combine_body(p0_ref, p1_ref, x_ref, w_ref, o_ref):
    o_ref[...] = (p0_ref[...] + p1_ref[...]
                  + jnp.dot(x_ref[...], w_ref[...],
                            preferred_element_type=jnp.float32))


@jax.jit
def _tc_combine(p0, p1, x, w):
    blk = 1000
    return pl.pallas_call(
        _combine_body,
        grid=(N // blk,),
        in_specs=[
            pl.BlockSpec((blk, D), lambda i: (i, 0)),
            pl.BlockSpec((blk, D), lambda i: (i, 0)),
            pl.BlockSpec((blk, D), lambda i: (i, 0)),
            pl.BlockSpec((D, D), lambda i: (0, 0)),
        ],
        out_specs=pl.BlockSpec((blk, D), lambda i: (i, 0)),
        out_shape=jax.ShapeDtypeStruct((N, D), jnp.float32),
    )(p0, p1, x, w)


def kernel(x, edge_index, loop_weight):
    src2d = edge_index[0].reshape(E // K, K)
    dst2d = edge_index[1].reshape(E // K, K)
    parts = _sc_scatter(x, src2d, dst2d)
    return _tc_combine(parts[0], parts[1], x, loop_weight)


# speedup vs baseline: 8.7312x; 8.7312x over previous
import jax
import jax.numpy as jnp
from jax import lax
from jax.experimental import pallas as pl
from jax.experimental.pallas import tpu as pltpu
from jax.experimental.pallas import tpu_sc as plsc

N = 10000
E = 320000
D = 128

NC = 2
NS = 16
K = 125
NPAD = 10240
EDGES_PER_TILE = E // (NC * NS)
CHUNKS_PER_TILE = EDGES_PER_TILE // K
ROWS_PER_TILE = NPAD // NS
ZROWS = 64


def _sc_scatter_kernel(x_hbm, src_hbm, dst_hbm, out0_hbm, out1_hbm,
                       src_v, dst_v, rows_v, zero_v, acc_shared, gsem):
    c = lax.axis_index("c")
    s = lax.axis_index("s")
    chunk0 = (c * NS + s) * CHUNKS_PER_TILE

    pltpu.sync_copy(src_hbm.at[pl.ds(chunk0, CHUNKS_PER_TILE)], src_v)
    pltpu.sync_copy(dst_hbm.at[pl.ds(chunk0, CHUNKS_PER_TILE)], dst_v)

    def _zrow(i, carry):
        for j in range(D // 16):
            zero_v[i, pl.ds(j * 16, 16)] = jnp.zeros((16,), jnp.float32)
        return carry
    lax.fori_loop(0, ZROWS, _zrow, 0)
    for r in range(ROWS_PER_TILE // ZROWS):
        pltpu.sync_copy(
            zero_v, acc_shared.at[pl.ds(s * ROWS_PER_TILE + r * ZROWS, ZROWS)])
    plsc.subcore_barrier()

    def _chunk(j, carry):
        pltpu.async_copy(x_hbm.at[src_v.at[j]], rows_v, gsem).wait()
        pltpu.sync_copy(rows_v, acc_shared.at[dst_v.at[j]], add=True)
        return carry
    lax.fori_loop(0, CHUNKS_PER_TILE, _chunk, 0)
    plsc.subcore_barrier()

    row0 = s * ROWS_PER_TILE
    acc_slice = acc_shared.at[pl.ds(row0, ROWS_PER_TILE)]

    @pl.when(c == 0)
    def _():
        pltpu.sync_copy(acc_slice, out0_hbm.at[pl.ds(row0, ROWS_PER_TILE)])

    @pl.when(c == 1)
    def _():
        pltpu.sync_copy(acc_slice, out1_hbm.at[pl.ds(row0, ROWS_PER_TILE)])


@jax.jit
def _sc_scatter(x, src2d, dst2d):
    return pl.kernel(
        _sc_scatter_kernel,
        out_type=(jax.ShapeDtypeStruct((NPAD, D), jnp.float32),
                  jax.ShapeDtypeStruct((NPAD, D), jnp.float32)),
        mesh=plsc.VectorSubcoreMesh(core_axis_name="c", subcore_axis_name="s"),
        scratch_types=[
            pltpu.VMEM((CHUNKS_PER_TILE, K), jnp.int32),
            pltpu.VMEM((CHUNKS_PER_TILE, K), jnp.int32),
            pltpu.VMEM((K, D), jnp.float32),
            pltpu.VMEM((ZROWS, D), jnp.float32),
            pltpu.VMEM_SHARED((NPAD, D), jnp.float32),
            pltpu.SemaphoreType.DMA,
        ],
    )(x, src2d, dst2d)


def _combine_body(p0_ref, p1_ref, x_ref, w_ref, o_ref):
    o_ref[...] = (p0_ref[...] + p1_ref[...]
                  + jnp.dot(x_ref[...], w_ref[...],
                            preferred_element_type=jnp.float32))


@jax.jit
def _tc_combine(p0, p1, x, w):
    blk = 1000
    return pl.pallas_call(
        _combine_body,
        grid=(N // blk,),
        in_specs=[
            pl.BlockSpec((blk, D), lambda i: (i, 0)),
            pl.BlockSpec((blk, D), lambda i: (i, 0)),
            pl.BlockSpec((blk, D), lambda i: (i, 0)),
            pl.BlockSpec((D, D), lambda i: (0, 0)),
        ],
        out_specs=pl.BlockSpec((blk, D), lambda i: (i, 0)),
        out_shape=jax.ShapeDtypeStruct((N, D), jnp.float32),
    )(p0, p1, x, w)


def kernel(x, edge_index, loop_weight):
    src2d = edge_index[0].reshape(E // K, K)
    dst2d = edge_index[1].reshape(E // K, K)
    p0, p1 = _sc_scatter(x, src2d, dst2d)
    return _tc_combine(p0, p1, x, loop_weight)
